# baseline (speedup 1.0000x reference)
import functools

import jax
import jax.numpy as jnp
from jax import lax
from jax.experimental import pallas as pl
from jax.experimental.pallas import tpu as pltpu
from jax.experimental.pallas import tpu_sc as plsc

B = 16384
EMB = 16
NROWS = 1000000
NCOLS = 100000
NC = 2
NS = 16
NW = NC * NS
BPW = B // NW
CHUNK = 128
NCH = BPW // CHUNK


@functools.lru_cache(maxsize=None)
def _make_sc_gather():
    mesh = plsc.VectorSubcoreMesh(
        core_axis_name="c", subcore_axis_name="s", num_cores=NC, num_subcores=NS
    )

    @functools.partial(
        pl.kernel,
        out_type=[
            jax.ShapeDtypeStruct((EMB, B), jnp.float32),
            jax.ShapeDtypeStruct((EMB, B), jnp.float32),
        ],
        mesh=mesh,
        scratch_types=[
            pltpu.VMEM((NCH, CHUNK), jnp.int32),
            pltpu.VMEM((NCH, CHUNK), jnp.int32),
            pltpu.VMEM((EMB, BPW), jnp.float32),
            pltpu.VMEM((EMB, BPW), jnp.float32),
            pltpu.SemaphoreType.DMA,
        ],
        compiler_params=pltpu.CompilerParams(use_tc_tiling_on_sc=False),
    )
    def sc_gather(row_tabT, col_tabT, ridx, cidx, rT_out, cT_out,
                  ridx_v, cidx_v, rT_v, cT_v, sem):
        wid = lax.axis_index("s") * NC + lax.axis_index("c")
        base = wid * BPW
        pltpu.sync_copy(ridx.at[wid], ridx_v)
        pltpu.sync_copy(cidx.at[wid], cidx_v)

        def chunk_body(ch, _):
            copies = []
            for j in range(EMB):
                copies.append(pltpu.async_copy(
                    row_tabT.at[j].at[ridx_v.at[ch]],
                    rT_v.at[j, pl.ds(ch * CHUNK, CHUNK)], sem))
                copies.append(pltpu.async_copy(
                    col_tabT.at[j].at[cidx_v.at[ch]],
                    cT_v.at[j, pl.ds(ch * CHUNK, CHUNK)], sem))
            for c in copies:
                c.wait()
            return 0

        lax.fori_loop(0, NCH, chunk_body, 0)
        pltpu.sync_copy(rT_v, rT_out.at[:, pl.ds(base, BPW)])
        pltpu.sync_copy(cT_v, cT_out.at[:, pl.ds(base, BPW)])

    return sc_gather


BLK = 2048


def _mlp_body(xT_ref, reT_ref, ceT_ref,
              W1T_ref, b1_ref, g1_ref, be1_ref,
              W2T_ref, b2_ref, g2_ref, be2_ref, W3_ref, b3_ref, o_ref):
    xT = xT_ref[...]
    reT = reT_ref[...]
    ceT = ceT_ref[...]
    W1T = W1T_ref[...]
    h = (W1T[:, 0:1] * xT[0:1, :] + W1T[:, 1:2] * xT[1:2, :]
         + jnp.dot(W1T[:, 2:2 + EMB], reT, preferred_element_type=jnp.float32,
                   precision=lax.Precision.HIGHEST)
         + jnp.dot(W1T[:, 2 + EMB:], ceT, preferred_element_type=jnp.float32,
                   precision=lax.Precision.HIGHEST)
         + b1_ref[...])
    h = _layernorm_gelu_t(h, g1_ref[...], be1_ref[...])
    h = jnp.dot(W2T_ref[...], h, preferred_element_type=jnp.float32,
                precision=lax.Precision.HIGHEST) + b2_ref[...]
    h = _layernorm_gelu_t(h, g2_ref[...], be2_ref[...])
    o_ref[...] = jnp.sum(h * W3_ref[...], axis=0, keepdims=True) + b3_ref[...]


def _layernorm_gelu_t(h, g, b, eps=1e-5):
    mu = jnp.mean(h, axis=0, keepdims=True)
    var = jnp.mean((h - mu) ** 2, axis=0, keepdims=True)
    h = (h - mu) / jnp.sqrt(var + eps) * g + b
    return h * 0.5 * (1.0 + lax.erf(h * (2.0 ** -0.5)))


def kernel(x, row_idx, col_idx, row_table, col_table,
           W1, b1, g1, be1, W2, b2, g2, be2, W3, b3):
    ridx = row_idx.astype(jnp.int32).reshape(NW, NCH, CHUNK)
    cidx = col_idx.astype(jnp.int32).reshape(NW, NCH, CHUNK)
    reT, ceT = _make_sc_gather()(row_table.T, col_table.T, ridx, cidx)

    grid = (B // BLK,)
    full = lambda i: (0, 0)
    batch = lambda i: (0, i)
    outT = pl.pallas_call(
        _mlp_body,
        grid=grid,
        in_specs=[
            pl.BlockSpec((2, BLK), batch),
            pl.BlockSpec((EMB, BLK), batch),
            pl.BlockSpec((EMB, BLK), batch),
            pl.BlockSpec((32, 2 + 2 * EMB), full),
            pl.BlockSpec((32, 1), full),
            pl.BlockSpec((32, 1), full),
            pl.BlockSpec((32, 1), full),
            pl.BlockSpec((16, 32), full),
            pl.BlockSpec((16, 1), full),
            pl.BlockSpec((16, 1), full),
            pl.BlockSpec((16, 1), full),
            pl.BlockSpec((16, 1), full),
            pl.BlockSpec((1, 1), full),
        ],
        out_specs=pl.BlockSpec((1, BLK), batch),
        out_shape=jax.ShapeDtypeStruct((1, B), jnp.float32),
    )(x.T, reT, ceT, W1.T,
      b1.reshape(32, 1), g1.reshape(32, 1), be1.reshape(32, 1),
      W2.T, b2.reshape(16, 1), g2.reshape(16, 1), be2.reshape(16, 1),
      W3, b3.reshape(1, 1))
    return outT.reshape(B, 1)


# speedup vs baseline: 11.8627x; 11.8627x over previous
import functools

import jax
import jax.numpy as jnp
from jax import lax
from jax.experimental import pallas as pl
from jax.experimental.pallas import tpu as pltpu
from jax.experimental.pallas import tpu_sc as plsc

B = 16384
EMB = 16
NROWS = 1000000
NCOLS = 100000
NC = 2
NS = 16
NW = NC * NS
BPW = B // NW
CHUNK = 128
NCH = BPW // CHUNK


def _delane_body(in_ref, *out_refs):
    x = in_ref[...]
    for j in range(EMB):
        out_refs[j][...] = x[j, :]


def _delane(tabT, n, ch):
    grid = ((n + ch - 1) // ch,)
    return pl.pallas_call(
        _delane_body,
        grid=grid,
        in_specs=[pl.BlockSpec((EMB, ch), lambda i: (0, i))],
        out_specs=[pl.BlockSpec((ch,), lambda i: (i,))] * EMB,
        out_shape=[jax.ShapeDtypeStruct((n,), jnp.float32)] * EMB,
    )(tabT)


@functools.lru_cache(maxsize=None)
def _make_sc_gather():
    mesh = plsc.VectorSubcoreMesh(
        core_axis_name="c", subcore_axis_name="s", num_cores=NC, num_subcores=NS
    )

    @functools.partial(
        pl.kernel,
        out_type=[
            jax.ShapeDtypeStruct((EMB, B), jnp.float32),
            jax.ShapeDtypeStruct((EMB, B), jnp.float32),
        ],
        mesh=mesh,
        scratch_types=[
            pltpu.VMEM((NCH, CHUNK), jnp.int32),
            pltpu.VMEM((NCH, CHUNK), jnp.int32),
            pltpu.VMEM((EMB, BPW), jnp.float32),
            pltpu.VMEM((EMB, BPW), jnp.float32),
            pltpu.SemaphoreType.DMA,
        ],
        compiler_params=pltpu.CompilerParams(use_tc_tiling_on_sc=False),
    )
    def sc_gather(*refs):
        rt = refs[0:EMB]
        ct = refs[EMB:2 * EMB]
        ridx, cidx, rT_out, cT_out = refs[2 * EMB:2 * EMB + 4]
        ridx_v, cidx_v, rT_v, cT_v, sem = refs[2 * EMB + 4:]
        wid = lax.axis_index("s") * NC + lax.axis_index("c")
        base = wid * BPW
        pltpu.sync_copy(ridx.at[wid], ridx_v)
        pltpu.sync_copy(cidx.at[wid], cidx_v)

        def chunk_body(ch, _):
            copies = []
            for j in range(EMB):
                copies.append(pltpu.async_copy(
                    rt[j].at[ridx_v.at[ch]],
                    rT_v.at[j, pl.ds(ch * CHUNK, CHUNK)], sem))
                copies.append(pltpu.async_copy(
                    ct[j].at[cidx_v.at[ch]],
                    cT_v.at[j, pl.ds(ch * CHUNK, CHUNK)], sem))
            for c in copies:
                c.wait()
            return 0

        lax.fori_loop(0, NCH, chunk_body, 0)
        pltpu.sync_copy(rT_v, rT_out.at[:, pl.ds(base, BPW)])
        pltpu.sync_copy(cT_v, cT_out.at[:, pl.ds(base, BPW)])

    return sc_gather


BLK = 2048


def _mlp_body(xT_ref, reT_ref, ceT_ref,
              W1T_ref, b1_ref, g1_ref, be1_ref,
              W2T_ref, b2_ref, g2_ref, be2_ref, W3_ref, b3_ref, o_ref):
    xT = xT_ref[...]
    reT = reT_ref[...]
    ceT = ceT_ref[...]
    W1T = W1T_ref[...]
    h = (W1T[:, 0:1] * xT[0:1, :] + W1T[:, 1:2] * xT[1:2, :]
         + jnp.dot(W1T[:, 2:2 + EMB], reT, preferred_element_type=jnp.float32,
                   precision=lax.Precision.HIGHEST)
         + jnp.dot(W1T[:, 2 + EMB:], ceT, preferred_element_type=jnp.float32,
                   precision=lax.Precision.HIGHEST)
         + b1_ref[...])
    h = _layernorm_gelu_t(h, g1_ref[...], be1_ref[...])
    h = jnp.dot(W2T_ref[...], h, preferred_element_type=jnp.float32,
                precision=lax.Precision.HIGHEST) + b2_ref[...]
    h = _layernorm_gelu_t(h, g2_ref[...], be2_ref[...])
    o_ref[...] = jnp.sum(h * W3_ref[...], axis=0, keepdims=True) + b3_ref[...]


def _layernorm_gelu_t(h, g, b, eps=1e-5):
    mu = jnp.mean(h, axis=0, keepdims=True)
    var = jnp.mean((h - mu) ** 2, axis=0, keepdims=True)
    h = (h - mu) / jnp.sqrt(var + eps) * g + b
    return h * 0.5 * (1.0 + lax.erf(h * (2.0 ** -0.5)))


def kernel(x, row_idx, col_idx, row_table, col_table,
           W1, b1, g1, be1, W2, b2, g2, be2, W3, b3):
    ridx = row_idx.astype(jnp.int32).reshape(NW, NCH, CHUNK)
    cidx = col_idx.astype(jnp.int32).reshape(NW, NCH, CHUNK)
    rts = _delane(row_table.T, NROWS, 40960)
    cts = _delane(col_table.T, NCOLS, 12288)
    reT, ceT = _make_sc_gather()(*rts, *cts, ridx, cidx)

    grid = (B // BLK,)
    full = lambda i: (0, 0)
    batch = lambda i: (0, i)
    outT = pl.pallas_call(
        _mlp_body,
        grid=grid,
        in_specs=[
            pl.BlockSpec((2, BLK), batch),
            pl.BlockSpec((EMB, BLK), batch),
            pl.BlockSpec((EMB, BLK), batch),
            pl.BlockSpec((32, 2 + 2 * EMB), full),
            pl.BlockSpec((32, 1), full),
            pl.BlockSpec((32, 1), full),
            pl.BlockSpec((32, 1), full),
            pl.BlockSpec((16, 32), full),
            pl.BlockSpec((16, 1), full),
            pl.BlockSpec((16, 1), full),
            pl.BlockSpec((16, 1), full),
            pl.BlockSpec((16, 1), full),
            pl.BlockSpec((1, 1), full),
        ],
        out_specs=pl.BlockSpec((1, BLK), batch),
        out_shape=jax.ShapeDtypeStruct((1, B), jnp.float32),
    )(x.T, reT, ceT, W1.T,
      b1.reshape(32, 1), g1.reshape(32, 1), be1.reshape(32, 1),
      W2.T, b2.reshape(16, 1), g2.reshape(16, 1), be2.reshape(16, 1),
      W3, b3.reshape(1, 1))
    return outT.reshape(B, 1)


# speedup vs baseline: 12.1493x; 1.0242x over previous
import functools

import jax
import jax.numpy as jnp
from jax import lax
from jax.experimental import pallas as pl
from jax.experimental.pallas import tpu as pltpu
from jax.experimental.pallas import tpu_sc as plsc

B = 16384
EMB = 16
NROWS = 1000000
NCOLS = 100000
NC = 2
NS = 16
NW = NC * NS
BPW = B // NW
CHUNK = 128
NCH = BPW // CHUNK


def _delane_body(in_ref, *out_refs):
    x = in_ref[...]
    for j in range(EMB):
        out_refs[j][...] = x[j, :]


def _delane(tabT, n, ch):
    grid = ((n + ch - 1) // ch,)
    return pl.pallas_call(
        _delane_body,
        grid=grid,
        in_specs=[pl.BlockSpec((EMB, ch), lambda i: (0, i))],
        out_specs=[pl.BlockSpec((ch,), lambda i: (i,))] * EMB,
        out_shape=[jax.ShapeDtypeStruct((n,), jnp.float32)] * EMB,
    )(tabT)


@functools.lru_cache(maxsize=None)
def _make_sc_gather():
    mesh = plsc.VectorSubcoreMesh(
        core_axis_name="c", subcore_axis_name="s", num_cores=NC, num_subcores=NS
    )

    @functools.partial(
        pl.kernel,
        out_type=jax.ShapeDtypeStruct((EMB, B), jnp.float32),
        mesh=mesh,
        scratch_types=[
            pltpu.VMEM((NCH, CHUNK), jnp.int32),
            pltpu.VMEM((EMB, BPW), jnp.float32),
            pltpu.SemaphoreType.DMA,
        ],
        compiler_params=pltpu.CompilerParams(use_tc_tiling_on_sc=False),
    )
    def sc_gather(*refs):
        tab = refs[0:EMB]
        idx, out = refs[EMB], refs[EMB + 1]
        idx_v, rows_v, sem = refs[EMB + 2:]
        wid = lax.axis_index("s") * NC + lax.axis_index("c")
        base = wid * BPW
        pltpu.sync_copy(idx.at[wid], idx_v)

        def chunk_body(ch, _):
            copies = []
            for j in range(EMB):
                copies.append(pltpu.async_copy(
                    tab[j].at[idx_v.at[ch]],
                    rows_v.at[j, pl.ds(ch * CHUNK, CHUNK)], sem))
            for c in copies:
                c.wait()
            return 0

        lax.fori_loop(0, NCH, chunk_body, 0)
        pltpu.sync_copy(rows_v, out.at[:, pl.ds(base, BPW)])

    return sc_gather


BLK = 2048


def _mlp_body(xT_ref, reT_ref, ceT_ref,
              W1T_ref, b1_ref, g1_ref, be1_ref,
              W2T_ref, b2_ref, g2_ref, be2_ref, W3_ref, b3_ref, o_ref):
    xT = xT_ref[...]
    reT = reT_ref[...]
    ceT = ceT_ref[...]
    W1T = W1T_ref[...]
    h = (W1T[:, 0:1] * xT[0:1, :] + W1T[:, 1:2] * xT[1:2, :]
         + jnp.dot(W1T[:, 2:2 + EMB], reT, preferred_element_type=jnp.float32,
                   precision=lax.Precision.HIGHEST)
         + jnp.dot(W1T[:, 2 + EMB:], ceT, preferred_element_type=jnp.float32,
                   precision=lax.Precision.HIGHEST)
         + b1_ref[...])
    h = _layernorm_gelu_t(h, g1_ref[...], be1_ref[...])
    h = jnp.dot(W2T_ref[...], h, preferred_element_type=jnp.float32,
                precision=lax.Precision.HIGHEST) + b2_ref[...]
    h = _layernorm_gelu_t(h, g2_ref[...], be2_ref[...])
    o_ref[...] = jnp.sum(h * W3_ref[...], axis=0, keepdims=True) + b3_ref[...]


def _layernorm_gelu_t(h, g, b, eps=1e-5):
    mu = jnp.mean(h, axis=0, keepdims=True)
    var = jnp.mean((h - mu) ** 2, axis=0, keepdims=True)
    h = (h - mu) / jnp.sqrt(var + eps) * g + b
    return h * 0.5 * (1.0 + lax.erf(h * (2.0 ** -0.5)))


def kernel(x, row_idx, col_idx, row_table, col_table,
           W1, b1, g1, be1, W2, b2, g2, be2, W3, b3):
    ridx = row_idx.astype(jnp.int32).reshape(NW, NCH, CHUNK)
    cidx = col_idx.astype(jnp.int32).reshape(NW, NCH, CHUNK)
    cts = _delane(col_table.T, NCOLS, 12288)
    ceT = _make_sc_gather()(*cts, cidx)
    rts = _delane(row_table.T, NROWS, 40960)
    reT = _make_sc_gather()(*rts, ridx)

    grid = (B // BLK,)
    full = lambda i: (0, 0)
    batch = lambda i: (0, i)
    outT = pl.pallas_call(
        _mlp_body,
        grid=grid,
        in_specs=[
            pl.BlockSpec((2, BLK), batch),
            pl.BlockSpec((EMB, BLK), batch),
            pl.BlockSpec((EMB, BLK), batch),
            pl.BlockSpec((32, 2 + 2 * EMB), full),
            pl.BlockSpec((32, 1), full),
            pl.BlockSpec((32, 1), full),
            pl.BlockSpec((32, 1), full),
            pl.BlockSpec((16, 32), full),
            pl.BlockSpec((16, 1), full),
            pl.BlockSpec((16, 1), full),
            pl.BlockSpec((16, 1), full),
            pl.BlockSpec((16, 1), full),
            pl.BlockSpec((1, 1), full),
        ],
        out_specs=pl.BlockSpec((1, BLK), batch),
        out_shape=jax.ShapeDtypeStruct((1, B), jnp.float32),
    )(x.T, reT, ceT, W1.T,
      b1.reshape(32, 1), g1.reshape(32, 1), be1.reshape(32, 1),
      W2.T, b2.reshape(16, 1), g2.reshape(16, 1), be2.reshape(16, 1),
      W3, b3.reshape(1, 1))
    return outT.reshape(B, 1)


# speedup vs baseline: 12.7726x; 1.0513x over previous
import functools

import jax
import jax.numpy as jnp
from jax import lax
from jax.experimental import pallas as pl
from jax.experimental.pallas import tpu as pltpu
from jax.experimental.pallas import tpu_sc as plsc

B = 16384
EMB = 16
NROWS = 1000000
NCOLS = 100000
NC = 2
NS = 16
NW = NC * NS
BPW = B // NW
CHUNK = 128
NCH = BPW // CHUNK


def _delane_body(*refs):
    x = refs[0][...]
    for j in range(EMB):
        refs[-EMB + j][...] = x[j, :]


def _delane(tabT, n, ch, dep=None):
    grid = ((n + ch - 1) // ch,)
    in_specs = [pl.BlockSpec((EMB, ch), lambda i: (0, i))]
    args = [tabT]
    if dep is not None:
        in_specs.append(pl.BlockSpec((1024,), lambda i: (0,)))
        args.append(dep)
    return pl.pallas_call(
        _delane_body,
        grid=grid,
        in_specs=in_specs,
        out_specs=[pl.BlockSpec((ch,), lambda i: (i,))] * EMB,
        out_shape=[jax.ShapeDtypeStruct((n,), jnp.float32)] * EMB,
    )(*args)


@functools.lru_cache(maxsize=None)
def _make_sc_gather():
    mesh = plsc.VectorSubcoreMesh(
        core_axis_name="c", subcore_axis_name="s", num_cores=NC, num_subcores=NS
    )

    @functools.partial(
        pl.kernel,
        out_type=jax.ShapeDtypeStruct((EMB, B), jnp.float32),
        mesh=mesh,
        scratch_types=[
            pltpu.VMEM((NCH, CHUNK), jnp.int32),
            pltpu.VMEM((EMB, BPW), jnp.float32),
            pltpu.SemaphoreType.DMA,
        ],
        compiler_params=pltpu.CompilerParams(use_tc_tiling_on_sc=False),
    )
    def sc_gather(*refs):
        tab = refs[0:EMB]
        idx, out = refs[EMB], refs[EMB + 1]
        idx_v, rows_v, sem = refs[EMB + 2:]
        wid = lax.axis_index("s") * NC + lax.axis_index("c")
        base = wid * BPW
        pltpu.sync_copy(idx.at[wid], idx_v)

        def chunk_body(ch, _):
            copies = []
            for j in range(EMB):
                copies.append(pltpu.async_copy(
                    tab[j].at[idx_v.at[ch]],
                    rows_v.at[j, pl.ds(ch * CHUNK, CHUNK)], sem))
            for c in copies:
                c.wait()
            return 0

        lax.fori_loop(0, NCH, chunk_body, 0)
        pltpu.sync_copy(rows_v, out.at[:, pl.ds(base, BPW)])

    return sc_gather


BLK = 2048


def _mlp_body(xT_ref, reT_ref, ceT_ref,
              W1T_ref, b1_ref, g1_ref, be1_ref,
              W2T_ref, b2_ref, g2_ref, be2_ref, W3_ref, b3_ref, o_ref):
    xT = xT_ref[...]
    reT = reT_ref[...]
    ceT = ceT_ref[...]
    W1T = W1T_ref[...]
    h = (W1T[:, 0:1] * xT[0:1, :] + W1T[:, 1:2] * xT[1:2, :]
         + jnp.dot(W1T[:, 2:2 + EMB], reT, preferred_element_type=jnp.float32,
                   precision=lax.Precision.HIGHEST)
         + jnp.dot(W1T[:, 2 + EMB:], ceT, preferred_element_type=jnp.float32,
                   precision=lax.Precision.HIGHEST)
         + b1_ref[...])
    h = _layernorm_gelu_t(h, g1_ref[...], be1_ref[...])
    h = jnp.dot(W2T_ref[...], h, preferred_element_type=jnp.float32,
                precision=lax.Precision.HIGHEST) + b2_ref[...]
    h = _layernorm_gelu_t(h, g2_ref[...], be2_ref[...])
    o_ref[...] = jnp.sum(h * W3_ref[...], axis=0, keepdims=True) + b3_ref[...]


def _layernorm_gelu_t(h, g, b, eps=1e-5):
    mu = jnp.mean(h, axis=0, keepdims=True)
    var = jnp.mean((h - mu) ** 2, axis=0, keepdims=True)
    h = (h - mu) / jnp.sqrt(var + eps) * g + b
    return h * 0.5 * (1.0 + lax.erf(h * (2.0 ** -0.5)))


def kernel(x, row_idx, col_idx, row_table, col_table,
           W1, b1, g1, be1, W2, b2, g2, be2, W3, b3):
    ridx = row_idx.astype(jnp.int32).reshape(NW, NCH, CHUNK)
    cidx = col_idx.astype(jnp.int32).reshape(NW, NCH, CHUNK)
    cts = _delane(col_table.T, NCOLS, 12288)
    ceT = _make_sc_gather()(*cts, cidx)
    rts = _delane(row_table.T, NROWS, 40960, dep=cts[0])
    reT = _make_sc_gather()(*rts, ridx)

    grid = (B // BLK,)
    full = lambda i: (0, 0)
    batch = lambda i: (0, i)
    outT = pl.pallas_call(
        _mlp_body,
        grid=grid,
        in_specs=[
            pl.BlockSpec((2, BLK), batch),
            pl.BlockSpec((EMB, BLK), batch),
            pl.BlockSpec((EMB, BLK), batch),
            pl.BlockSpec((32, 2 + 2 * EMB), full),
            pl.BlockSpec((32, 1), full),
            pl.BlockSpec((32, 1), full),
            pl.BlockSpec((32, 1), full),
            pl.BlockSpec((16, 32), full),
            pl.BlockSpec((16, 1), full),
            pl.BlockSpec((16, 1), full),
            pl.BlockSpec((16, 1), full),
            pl.BlockSpec((16, 1), full),
            pl.BlockSpec((1, 1), full),
        ],
        out_specs=pl.BlockSpec((1, BLK), batch),
        out_shape=jax.ShapeDtypeStruct((1, B), jnp.float32),
    )(x.T, reT, ceT, W1.T,
      b1.reshape(32, 1), g1.reshape(32, 1), be1.reshape(32, 1),
      W2.T, b2.reshape(16, 1), g2.reshape(16, 1), be2.reshape(16, 1),
      W3, b3.reshape(1, 1))
    return outT.reshape(B, 1)


# speedup vs baseline: 13.7158x; 1.0738x over previous
import functools

import jax
import jax.numpy as jnp
from jax import lax
from jax.experimental import pallas as pl
from jax.experimental.pallas import tpu as pltpu
from jax.experimental.pallas import tpu_sc as plsc

B = 16384
EMB = 16
NROWS = 1000000
NCOLS = 100000
NC = 2
NS = 16
NW = NC * NS
BPW = B // NW
CHUNK = 128
NCH = BPW // CHUNK


def _delane_body(*refs):
    x = refs[0][...]
    for j in range(EMB):
        refs[-EMB + j][...] = x[j, :]


def _delane(tabT, n, ch, dep=None):
    grid = ((n + ch - 1) // ch,)
    in_specs = [pl.BlockSpec((EMB, ch), lambda i: (0, i))]
    args = [tabT]
    if dep is not None:
        in_specs.append(pl.BlockSpec((1024,), lambda i: (0,)))
        args.append(dep)
    return pl.pallas_call(
        _delane_body,
        grid=grid,
        in_specs=in_specs,
        out_specs=[pl.BlockSpec((ch,), lambda i: (i,))] * EMB,
        out_shape=[jax.ShapeDtypeStruct((n,), jnp.float32)] * EMB,
    )(*args)


@functools.lru_cache(maxsize=None)
def _make_sc_gather():
    mesh = plsc.VectorSubcoreMesh(
        core_axis_name="c", subcore_axis_name="s", num_cores=NC, num_subcores=NS
    )

    @functools.partial(
        pl.kernel,
        out_type=jax.ShapeDtypeStruct((EMB, B), jnp.float32),
        mesh=mesh,
        scratch_types=[
            pltpu.VMEM((NCH, CHUNK), jnp.int32),
            pltpu.VMEM((EMB, BPW), jnp.float32),
            pltpu.SemaphoreType.DMA,
        ],
        compiler_params=pltpu.CompilerParams(use_tc_tiling_on_sc=False),
    )
    def sc_gather(*refs):
        tab = refs[0:EMB]
        idx, out = refs[EMB], refs[EMB + 1]
        idx_v, rows_v, sem = refs[EMB + 2:]
        wid = lax.axis_index("s") * NC + lax.axis_index("c")
        base = wid * BPW
        pltpu.sync_copy(idx.at[wid], idx_v)

        def chunk_body(ch, _):
            copies = []
            for j in range(EMB):
                copies.append(pltpu.async_copy(
                    tab[j].at[idx_v.at[ch]],
                    rows_v.at[j, pl.ds(ch * CHUNK, CHUNK)], sem))
            for c in copies:
                c.wait()
            return 0

        lax.fori_loop(0, NCH, chunk_body, 0)
        pltpu.sync_copy(rows_v, out.at[:, pl.ds(base, BPW)])

    return sc_gather


BLK = 2048


def _mlp_body(xT_ref, reT_ref, ceT_ref,
              W1T_ref, b1_ref, g1_ref, be1_ref,
              W2T_ref, b2_ref, g2_ref, be2_ref, W3_ref, b3_ref, o_ref):
    xT = xT_ref[...]
    reT = reT_ref[...]
    ceT = ceT_ref[...]
    W1T = W1T_ref[...]
    h = (W1T[:, 0:1] * xT[0:1, :] + W1T[:, 1:2] * xT[1:2, :]
         + jnp.dot(W1T[:, 2:2 + EMB], reT, preferred_element_type=jnp.float32,
                   precision=lax.Precision.HIGHEST)
         + jnp.dot(W1T[:, 2 + EMB:], ceT, preferred_element_type=jnp.float32,
                   precision=lax.Precision.HIGHEST)
         + b1_ref[...])
    h = _layernorm_gelu_t(h, g1_ref[...], be1_ref[...])
    h = jnp.dot(W2T_ref[...], h, preferred_element_type=jnp.float32,
                precision=lax.Precision.HIGHEST) + b2_ref[...]
    h = _layernorm_gelu_t(h, g2_ref[...], be2_ref[...])
    o_ref[...] = jnp.sum(h * W3_ref[...], axis=0, keepdims=True) + b3_ref[...]


def _layernorm_gelu_t(h, g, b, eps=1e-5):
    mu = jnp.mean(h, axis=0, keepdims=True)
    var = jnp.mean((h - mu) ** 2, axis=0, keepdims=True)
    h = (h - mu) / jnp.sqrt(var + eps) * g + b
    return h * 0.5 * (1.0 + lax.erf(h * (2.0 ** -0.5)))


def kernel(x, row_idx, col_idx, row_table, col_table,
           W1, b1, g1, be1, W2, b2, g2, be2, W3, b3):
    ridx = row_idx.astype(jnp.int32).reshape(NW, NCH, CHUNK)
    cidx = col_idx.astype(jnp.int32).reshape(NW, NCH, CHUNK)
    cts = _delane(col_table.T, NCOLS, 25600)
    ceT = _make_sc_gather()(*cts, cidx)
    rts = _delane(row_table.T, NROWS, 81920, dep=cts[0])
    reT = _make_sc_gather()(*rts, ridx)

    grid = (B // BLK,)
    full = lambda i: (0, 0)
    batch = lambda i: (0, i)
    outT = pl.pallas_call(
        _mlp_body,
        grid=grid,
        in_specs=[
            pl.BlockSpec((2, BLK), batch),
            pl.BlockSpec((EMB, BLK), batch),
            pl.BlockSpec((EMB, BLK), batch),
            pl.BlockSpec((32, 2 + 2 * EMB), full),
            pl.BlockSpec((32, 1), full),
            pl.BlockSpec((32, 1), full),
            pl.BlockSpec((32, 1), full),
            pl.BlockSpec((16, 32), full),
            pl.BlockSpec((16, 1), full),
            pl.BlockSpec((16, 1), full),
            pl.BlockSpec((16, 1), full),
            pl.BlockSpec((16, 1), full),
            pl.BlockSpec((1, 1), full),
        ],
        out_specs=pl.BlockSpec((1, BLK), batch),
        out_shape=jax.ShapeDtypeStruct((1, B), jnp.float32),
    )(x.T, reT, ceT, W1.T,
      b1.reshape(32, 1), g1.reshape(32, 1), be1.reshape(32, 1),
      W2.T, b2.reshape(16, 1), g2.reshape(16, 1), be2.reshape(16, 1),
      W3, b3.reshape(1, 1))
    return outT.reshape(B, 1)


# speedup vs baseline: 13.8846x; 1.0123x over previous
import functools

import jax
import jax.numpy as jnp
from jax import lax
from jax.experimental import pallas as pl
from jax.experimental.pallas import tpu as pltpu
from jax.experimental.pallas import tpu_sc as plsc

B = 16384
EMB = 16
NROWS = 1000000
NCOLS = 100000
NC = 2
NS = 16
NW = NC * NS
BPW = B // NW
CHUNK = 128
NCH = BPW // CHUNK


def _delane_body(*refs):
    x = refs[0][...]
    for j in range(EMB):
        refs[-EMB + j][...] = x[j, :]


def _delane(tabT, n, ch, dep=None):
    grid = ((n + ch - 1) // ch,)
    in_specs = [pl.BlockSpec((EMB, ch), lambda i: (0, i))]
    args = [tabT]
    if dep is not None:
        in_specs.append(pl.BlockSpec((1024,), lambda i: (0,)))
        args.append(dep)
    return pl.pallas_call(
        _delane_body,
        grid=grid,
        in_specs=in_specs,
        out_specs=[pl.BlockSpec((ch,), lambda i: (i,))] * EMB,
        out_shape=[jax.ShapeDtypeStruct((n,), jnp.float32)] * EMB,
    )(*args)


@functools.lru_cache(maxsize=None)
def _make_sc_gather():
    mesh = plsc.VectorSubcoreMesh(
        core_axis_name="c", subcore_axis_name="s", num_cores=NC, num_subcores=NS
    )

    @functools.partial(
        pl.kernel,
        out_type=jax.ShapeDtypeStruct((EMB, B), jnp.float32),
        mesh=mesh,
        scratch_types=[
            pltpu.VMEM((NCH, CHUNK), jnp.int32),
            pltpu.VMEM((EMB, BPW), jnp.float32),
            pltpu.SemaphoreType.DMA,
        ],
        compiler_params=pltpu.CompilerParams(use_tc_tiling_on_sc=False),
    )
    def sc_gather(*refs):
        tab = refs[0:EMB]
        idx, out = refs[EMB], refs[EMB + 1]
        idx_v, rows_v, sem = refs[EMB + 2:]
        wid = lax.axis_index("s") * NC + lax.axis_index("c")
        base = wid * BPW
        pltpu.sync_copy(idx.at[wid], idx_v)

        def chunk_body(ch, _):
            copies = []
            for j in range(EMB):
                copies.append(pltpu.async_copy(
                    tab[j].at[idx_v.at[ch]],
                    rows_v.at[j, pl.ds(ch * CHUNK, CHUNK)], sem))
            for c in copies:
                c.wait()
            return 0

        lax.fori_loop(0, NCH, chunk_body, 0)
        pltpu.sync_copy(rows_v, out.at[:, pl.ds(base, BPW)])

    return sc_gather


BLK = 2048


def _mlp_body(xT_ref, reT_ref, ceT_ref,
              W1T_ref, b1_ref, g1_ref, be1_ref,
              W2T_ref, b2_ref, g2_ref, be2_ref, W3_ref, b3_ref, o_ref):
    xT = xT_ref[...]
    reT = reT_ref[...]
    ceT = ceT_ref[...]
    W1T = W1T_ref[...]
    h = (W1T[:, 0:1] * xT[0:1, :] + W1T[:, 1:2] * xT[1:2, :]
         + jnp.dot(W1T[:, 2:2 + EMB], reT, preferred_element_type=jnp.float32,
                   precision=lax.Precision.HIGHEST)
         + jnp.dot(W1T[:, 2 + EMB:], ceT, preferred_element_type=jnp.float32,
                   precision=lax.Precision.HIGHEST)
         + b1_ref[...])
    h = _layernorm_gelu_t(h, g1_ref[...], be1_ref[...])
    h = jnp.dot(W2T_ref[...], h, preferred_element_type=jnp.float32,
                precision=lax.Precision.HIGHEST) + b2_ref[...]
    h = _layernorm_gelu_t(h, g2_ref[...], be2_ref[...])
    o_ref[...] = jnp.sum(h * W3_ref[...], axis=0, keepdims=True) + b3_ref[...]


def _layernorm_gelu_t(h, g, b, eps=1e-5):
    mu = jnp.mean(h, axis=0, keepdims=True)
    var = jnp.mean((h - mu) ** 2, axis=0, keepdims=True)
    h = (h - mu) / jnp.sqrt(var + eps) * g + b
    return h * 0.5 * (1.0 + lax.erf(h * (2.0 ** -0.5)))


def kernel(x, row_idx, col_idx, row_table, col_table,
           W1, b1, g1, be1, W2, b2, g2, be2, W3, b3):
    ridx = row_idx.astype(jnp.int32).reshape(NW, NCH, CHUNK)
    cidx = col_idx.astype(jnp.int32).reshape(NW, NCH, CHUNK)
    cts = _delane(col_table.T, NCOLS, 50176)
    ceT = _make_sc_gather()(*cts, cidx)
    rts = _delane(row_table.T, NROWS, 102400, dep=cts[0])
    reT = _make_sc_gather()(*rts, ridx)

    grid = (B // BLK,)
    full = lambda i: (0, 0)
    batch = lambda i: (0, i)
    outT = pl.pallas_call(
        _mlp_body,
        grid=grid,
        in_specs=[
            pl.BlockSpec((2, BLK), batch),
            pl.BlockSpec((EMB, BLK), batch),
            pl.BlockSpec((EMB, BLK), batch),
            pl.BlockSpec((32, 2 + 2 * EMB), full),
            pl.BlockSpec((32, 1), full),
            pl.BlockSpec((32, 1), full),
            pl.BlockSpec((32, 1), full),
            pl.BlockSpec((16, 32), full),
            pl.BlockSpec((16, 1), full),
            pl.BlockSpec((16, 1), full),
            pl.BlockSpec((16, 1), full),
            pl.BlockSpec((16, 1), full),
            pl.BlockSpec((1, 1), full),
        ],
        out_specs=pl.BlockSpec((1, BLK), batch),
        out_shape=jax.ShapeDtypeStruct((1, B), jnp.float32),
    )(x.T, reT, ceT, W1.T,
      b1.reshape(32, 1), g1.reshape(32, 1), be1.reshape(32, 1),
      W2.T, b2.reshape(16, 1), g2.reshape(16, 1), be2.reshape(16, 1),
      W3, b3.reshape(1, 1))
    return outT.reshape(B, 1)


# speedup vs baseline: 14.1480x; 1.0190x over previous
import functools

import jax
import jax.numpy as jnp
from jax import lax
from jax.experimental import pallas as pl
from jax.experimental.pallas import tpu as pltpu
from jax.experimental.pallas import tpu_sc as plsc

B = 16384
EMB = 16
NROWS = 1000000
NCOLS = 100000
NC = 2
NS = 16
NW = NC * NS
BPW = B // NW
CHUNK = 128
NCH = BPW // CHUNK


def _delane_body(*refs):
    x = refs[0][...]
    for j in range(EMB):
        refs[-EMB + j][...] = x[j, :]


def _delane(tabT, n, ch, dep=None):
    grid = ((n + ch - 1) // ch,)
    in_specs = [pl.BlockSpec((EMB, ch), lambda i: (0, i))]
    args = [tabT]
    if dep is not None:
        in_specs.append(pl.BlockSpec((1024,), lambda i: (0,)))
        args.append(dep)
    return pl.pallas_call(
        _delane_body,
        grid=grid,
        in_specs=in_specs,
        out_specs=[pl.BlockSpec((ch,), lambda i: (i,))] * EMB,
        out_shape=[jax.ShapeDtypeStruct((n,), jnp.float32)] * EMB,
    )(*args)


@functools.lru_cache(maxsize=None)
def _make_sc_gather():
    mesh = plsc.VectorSubcoreMesh(
        core_axis_name="c", subcore_axis_name="s", num_cores=NC, num_subcores=NS
    )

    @functools.partial(
        pl.kernel,
        out_type=jax.ShapeDtypeStruct((EMB, B), jnp.float32),
        mesh=mesh,
        scratch_types=[
            pltpu.VMEM((NCH, CHUNK), jnp.int32),
            pltpu.VMEM((EMB, BPW), jnp.float32),
            pltpu.SemaphoreType.DMA,
        ],
        compiler_params=pltpu.CompilerParams(use_tc_tiling_on_sc=False),
    )
    def sc_gather(*refs):
        tab = refs[0:EMB]
        idx, out = refs[EMB], refs[EMB + 1]
        idx_v, rows_v, sem = refs[EMB + 2:]
        wid = lax.axis_index("s") * NC + lax.axis_index("c")
        base = wid * BPW
        pltpu.sync_copy(idx.at[wid], idx_v)

        def chunk_body(ch, _):
            for j in range(EMB):
                pltpu.async_copy(
                    tab[j].at[idx_v.at[ch]],
                    rows_v.at[j, pl.ds(ch * CHUNK, CHUNK)], sem)
            return 0

        lax.fori_loop(0, NCH, chunk_body, 0)
        for j in range(EMB):
            pltpu.make_async_copy(
                tab[j].at[pl.ds(0, BPW)], rows_v.at[j], sem).wait()
        pltpu.sync_copy(rows_v, out.at[:, pl.ds(base, BPW)])

    return sc_gather


BLK = 2048


def _mlp_body(xT_ref, reT_ref, ceT_ref,
              W1T_ref, b1_ref, g1_ref, be1_ref,
              W2T_ref, b2_ref, g2_ref, be2_ref, W3_ref, b3_ref, o_ref):
    xT = xT_ref[...]
    reT = reT_ref[...]
    ceT = ceT_ref[...]
    W1T = W1T_ref[...]
    h = (W1T[:, 0:1] * xT[0:1, :] + W1T[:, 1:2] * xT[1:2, :]
         + jnp.dot(W1T[:, 2:2 + EMB], reT, preferred_element_type=jnp.float32,
                   precision=lax.Precision.HIGHEST)
         + jnp.dot(W1T[:, 2 + EMB:], ceT, preferred_element_type=jnp.float32,
                   precision=lax.Precision.HIGHEST)
         + b1_ref[...])
    h = _layernorm_gelu_t(h, g1_ref[...], be1_ref[...])
    h = jnp.dot(W2T_ref[...], h, preferred_element_type=jnp.float32,
                precision=lax.Precision.HIGHEST) + b2_ref[...]
    h = _layernorm_gelu_t(h, g2_ref[...], be2_ref[...])
    o_ref[...] = jnp.sum(h * W3_ref[...], axis=0, keepdims=True) + b3_ref[...]


def _layernorm_gelu_t(h, g, b, eps=1e-5):
    mu = jnp.mean(h, axis=0, keepdims=True)
    var = jnp.mean((h - mu) ** 2, axis=0, keepdims=True)
    h = (h - mu) / jnp.sqrt(var + eps) * g + b
    return h * 0.5 * (1.0 + lax.erf(h * (2.0 ** -0.5)))


def kernel(x, row_idx, col_idx, row_table, col_table,
           W1, b1, g1, be1, W2, b2, g2, be2, W3, b3):
    ridx = row_idx.astype(jnp.int32).reshape(NW, NCH, CHUNK)
    cidx = col_idx.astype(jnp.int32).reshape(NW, NCH, CHUNK)
    cts = _delane(col_table.T, NCOLS, 50176)
    ceT = _make_sc_gather()(*cts, cidx)
    rts = _delane(row_table.T, NROWS, 102400, dep=cts[0])
    reT = _make_sc_gather()(*rts, ridx)

    grid = (B // BLK,)
    full = lambda i: (0, 0)
    batch = lambda i: (0, i)
    outT = pl.pallas_call(
        _mlp_body,
        grid=grid,
        in_specs=[
            pl.BlockSpec((2, BLK), batch),
            pl.BlockSpec((EMB, BLK), batch),
            pl.BlockSpec((EMB, BLK), batch),
            pl.BlockSpec((32, 2 + 2 * EMB), full),
            pl.BlockSpec((32, 1), full),
            pl.BlockSpec((32, 1), full),
            pl.BlockSpec((32, 1), full),
            pl.BlockSpec((16, 32), full),
            pl.BlockSpec((16, 1), full),
            pl.BlockSpec((16, 1), full),
            pl.BlockSpec((16, 1), full),
            pl.BlockSpec((16, 1), full),
            pl.BlockSpec((1, 1), full),
        ],
        out_specs=pl.BlockSpec((1, BLK), batch),
        out_shape=jax.ShapeDtypeStruct((1, B), jnp.float32),
    )(x.T, reT, ceT, W1.T,
      b1.reshape(32, 1), g1.reshape(32, 1), be1.reshape(32, 1),
      W2.T, b2.reshape(16, 1), g2.reshape(16, 1), be2.reshape(16, 1),
      W3, b3.reshape(1, 1))
    return outT.reshape(B, 1)
